# baseline (speedup 1.0000x reference)
import functools

import jax
import jax.numpy as jnp
from jax import lax
from jax.experimental import pallas as pl
from jax.experimental.pallas import tpu as pltpu
from jax.experimental.pallas import tpu_sc as plsc

N_USERS = 5000
N_ITEMS = 5000
NUM_RATINGS = 5
D_IN = 128
AGG_UNITS = 256
OUT_UNITS = 128
NUM_BASIS = 2
E_ENC = 160000
E_DEC = 100000

NPAD = 5120
NLANE = 16
ACC_W = NPAD * NLANE

ENC_PER_TILE = E_ENC // 16

MSG_CH = 640
MSG_SUP = 6400
MSG_CHPS = MSG_SUP // MSG_CH
MSG_NSUP = E_ENC // MSG_SUP

E_DEC_PAD = 102400
DEC_PER_W = E_DEC_PAD // 32
DEC_CHUNK = 128
DEC_NCHUNK = DEC_PER_W // DEC_CHUNK

_SC_MESH = plsc.VectorSubcoreMesh(core_axis_name="c", subcore_axis_name="s")


@functools.partial(
    pl.kernel,
    out_type=jax.ShapeDtypeStruct((2, 16, NPAD), jnp.float32),
    mesh=_SC_MESH,
    scratch_types=[
        pltpu.VMEM((ENC_PER_TILE,), jnp.int32),
        pltpu.VMEM((ACC_W,), jnp.float32),
        pltpu.VMEM((NPAD,), jnp.float32),
    ],
    compiler_params=pltpu.CompilerParams(needs_layout_passes=False),
)
def _sc_degrees(didx_hbm, out_hbm, idx_v, acc_v, red_v):
    cid = lax.axis_index("c")
    tid = lax.axis_index("s")
    zeros = jnp.zeros((16,), jnp.float32)
    ones = jnp.ones((16,), jnp.float32)
    iota = lax.iota(jnp.int32, 16)

    @plsc.parallel_loop(0, ACC_W // 16, unroll=8)
    def zbody(i):
        acc_v[pl.ds(i * 16, 16)] = zeros

    pltpu.sync_copy(didx_hbm.at[cid, tid], idx_v)

    @plsc.parallel_loop(0, ENC_PER_TILE // 16, unroll=8)
    def body(g):
        s16 = idx_v[pl.ds(g * 16, 16)]
        plsc.addupdate_scatter(acc_v, [iota * NPAD + s16], ones)

    @plsc.parallel_loop(0, NPAD // 16, unroll=2)
    def rbody(i):
        v = acc_v[pl.ds(i * 16, 16)]
        for l in range(1, 16):
            v = v + acc_v[pl.ds(l * NPAD + i * 16, 16)]
        red_v[pl.ds(i * 16, 16)] = v

    pltpu.sync_copy(red_v, out_hbm.at[cid, tid])


@functools.partial(
    pl.kernel,
    out_type=jax.ShapeDtypeStruct((2, 16, ACC_W), jnp.float32),
    mesh=_SC_MESH,
    scratch_types=[
        pltpu.VMEM((MSG_SUP,), jnp.int32),
        pltpu.VMEM((MSG_SUP,), jnp.int32),
        pltpu.VMEM((MSG_CH, NLANE), jnp.float32),
        pltpu.VMEM((MSG_CH, NLANE), jnp.float32),
        pltpu.VMEM((ACC_W,), jnp.float32),
        pltpu.SemaphoreType.DMA,
        pltpu.SemaphoreType.DMA,
    ],
    compiler_params=pltpu.CompilerParams(needs_layout_passes=False,
                                         use_tc_tiling_on_sc=False),
)
def _sc_messages(htab_hbm, gidx_hbm, sidx_hbm, out_hbm,
                 gv, sv, rows0, rows1, acc_v, sem0, sem1):
    cid = lax.axis_index("c")
    tid = lax.axis_index("s")
    zeros = jnp.zeros((16,), jnp.float32)
    iota = lax.iota(jnp.int32, 16)

    @plsc.parallel_loop(0, ACC_W // 16, unroll=8)
    def zbody(i):
        acc_v[pl.ds(i * 16, 16)] = zeros

    tab = htab_hbm
    bufs = (rows0, rows1)
    sems = (sem0, sem1)

    def gather(c):
        return pltpu.async_copy(
            tab.at[gv.at[pl.ds(c * MSG_CH, MSG_CH)]],
            bufs[c % 2], sems[c % 2])

    def accumulate(c):
        rows = bufs[c % 2]

        @plsc.parallel_loop(0, MSG_CH // 16, unroll=8)
        def abody(g):
            base = c * MSG_CH + g * 16
            d16 = sv[pl.ds(base, 16)]
            for u in range(16):
                plsc.addupdate_scatter(
                    acc_v, [d16[u] + iota], rows[g * 16 + u])

    def sup(b, _):
        pltpu.sync_copy(gidx_hbm.at[tid, cid, b], gv)
        pltpu.sync_copy(sidx_hbm.at[cid, b], sv)
        cps = [gather(0)]
        for c in range(MSG_CHPS):
            if c + 1 < MSG_CHPS:
                cps.append(gather(c + 1))
            cps[c].wait()
            accumulate(c)
        return _

    lax.fori_loop(0, MSG_NSUP, sup, None)
    pltpu.sync_copy(acc_v, out_hbm.at[cid, tid])


@functools.partial(
    pl.kernel,
    out_type=(jax.ShapeDtypeStruct((E_DEC_PAD, OUT_UNITS), jnp.float32),
              jax.ShapeDtypeStruct((E_DEC_PAD, OUT_UNITS), jnp.float32)),
    mesh=_SC_MESH,
    scratch_types=[
        pltpu.VMEM((DEC_PER_W,), jnp.int32),
        pltpu.VMEM((DEC_PER_W,), jnp.int32),
        pltpu.VMEM((DEC_CHUNK, OUT_UNITS), jnp.float32),
        pltpu.VMEM((DEC_CHUNK, OUT_UNITS), jnp.float32),
        pltpu.SemaphoreType.DMA,
        pltpu.SemaphoreType.DMA,
    ],
)
def _sc_dec_gather(nutab_hbm, ivtab_hbm, sidx_hbm, didx_hbm,
                   ue_hbm, iv_hbm, sidx_v, didx_v, u_v, v_v, sem1, sem2):
    cid = lax.axis_index("c")
    tid = lax.axis_index("s")
    wid = tid * 2 + cid
    pltpu.sync_copy(sidx_hbm.at[wid], sidx_v)
    pltpu.sync_copy(didx_hbm.at[wid], didx_v)
    base = wid * DEC_PER_W

    def body(j, _):
        cu = pltpu.async_copy(
            nutab_hbm.at[sidx_v.at[pl.ds(j * DEC_CHUNK, DEC_CHUNK)]],
            u_v, sem1)
        cv = pltpu.async_copy(
            ivtab_hbm.at[didx_v.at[pl.ds(j * DEC_CHUNK, DEC_CHUNK)]],
            v_v, sem2)
        cu.wait()
        cv.wait()
        pltpu.sync_copy(u_v, ue_hbm.at[pl.ds(base + j * DEC_CHUNK, DEC_CHUNK)])
        pltpu.sync_copy(v_v, iv_hbm.at[pl.ds(base + j * DEC_CHUNK, DEC_CHUNK)])
        return _

    lax.fori_loop(0, DEC_NCHUNK, body, None)


def _prep_body(feat_ref, deg_ref, w_ref, out_ref):
    d = jnp.sum(deg_ref[0], axis=1, keepdims=True)
    c = lax.rsqrt(jnp.maximum(d[:N_USERS, :], 1.0))
    x = feat_ref[0] * c
    out_ref[...] = jnp.dot(x, w_ref[0], preferred_element_type=jnp.float32)


def _tc_prep(feats, dcol3, W):
    return pl.pallas_call(
        _prep_body,
        grid=(2, NUM_RATINGS),
        in_specs=[
            pl.BlockSpec((1, N_USERS, D_IN), lambda s, r: (s, 0, 0)),
            pl.BlockSpec((1, NPAD, 16), lambda s, r: (s, 0, 0)),
            pl.BlockSpec((1, D_IN, AGG_UNITS), lambda s, r: (r, 0, 0)),
        ],
        out_specs=pl.BlockSpec((N_USERS, AGG_UNITS),
                               lambda s, r: (s * NUM_RATINGS + r, 0)),
        out_shape=jax.ShapeDtypeStruct((2 * NUM_RATINGS * N_USERS, AGG_UNITS),
                                       jnp.float32),
    )(feats, dcol3, W)


def _mid_body(agg_ref, deg_ref, fcw_ref, fcb_ref, gnw_ref, gnb_ref,
              gms_ref, nu_ref, iv_ref):
    du = jnp.sum(deg_ref[0], axis=1, keepdims=True)
    di = jnp.sum(deg_ref[1], axis=1, keepdims=True)
    cu = lax.rsqrt(jnp.maximum(du[:N_USERS, :], 1.0))
    ci = lax.rsqrt(jnp.maximum(di[:N_ITEMS, :], 1.0))
    ua = agg_ref[1, :N_USERS, :] * cu
    ia = agg_ref[0, :N_ITEMS, :] * ci
    ua = jnp.where(ua >= 0, ua, 0.1 * ua)
    ia = jnp.where(ia >= 0, ia, 0.1 * ia)
    uo = jnp.dot(ua, fcw_ref[...], preferred_element_type=jnp.float32) + fcb_ref[...]
    io = jnp.dot(ia, fcw_ref[...], preferred_element_type=jnp.float32) + fcb_ref[...]
    n_tot = float(N_USERS + N_ITEMS)
    mean = (jnp.sum(uo, axis=0, keepdims=True)
            + jnp.sum(io, axis=0, keepdims=True)) / n_tot
    subu = uo - mean * gms_ref[...]
    subi = io - mean * gms_ref[...]
    var = (jnp.sum(subu * subu, axis=0, keepdims=True)
           + jnp.sum(subi * subi, axis=0, keepdims=True)) / n_tot
    inv_std = lax.rsqrt(var + 1e-6)
    nu = gnw_ref[...] * subu * inv_std + gnb_ref[...]
    ni = gnw_ref[...] * subi * inv_std + gnb_ref[...]
    nu_ref[...] = nu
    iv_ref[...] = ni


def _tc_mid(agg, dcol3, fc_w, fc_b, gn_weight, gn_bias, gn_mean_scale):
    return pl.pallas_call(
        _mid_body,
        out_shape=(jax.ShapeDtypeStruct((N_USERS, OUT_UNITS), jnp.float32),
                   jax.ShapeDtypeStruct((N_ITEMS, OUT_UNITS), jnp.float32)),
    )(agg, dcol3, fc_w, fc_b[None, :], gn_weight[None, :], gn_bias[None, :],
      gn_mean_scale[None, :])


_FIN_BLK = 2048


def _final_body(ue_ref, iv_ref, p_ref, cw_ref, out_ref):
    ue = ue_ref[...]
    iv = iv_ref[...]
    t0 = lax.dot_general(iv, p_ref[0], (((1,), (1,)), ((), ())),
                         preferred_element_type=jnp.float32)
    t1 = lax.dot_general(iv, p_ref[1], (((1,), (1,)), ((), ())),
                         preferred_element_type=jnp.float32)
    b0 = jnp.sum(ue * t0, axis=1, keepdims=True)
    b1 = jnp.sum(ue * t1, axis=1, keepdims=True)
    out_ref[...] = b0 * cw_ref[0, :][None, :] + b1 * cw_ref[1, :][None, :]


def _tc_final(ue, iv, P, combine_w):
    return pl.pallas_call(
        _final_body,
        grid=(E_DEC_PAD // _FIN_BLK,),
        in_specs=[
            pl.BlockSpec((_FIN_BLK, OUT_UNITS), lambda i: (i, 0)),
            pl.BlockSpec((_FIN_BLK, OUT_UNITS), lambda i: (i, 0)),
            pl.BlockSpec((NUM_BASIS, OUT_UNITS, OUT_UNITS), lambda i: (0, 0, 0)),
            pl.BlockSpec((NUM_BASIS, NUM_RATINGS), lambda i: (0, 0)),
        ],
        out_specs=pl.BlockSpec((_FIN_BLK, NUM_RATINGS), lambda i: (i, 0)),
        out_shape=jax.ShapeDtypeStruct((E_DEC_PAD, NUM_RATINGS), jnp.float32),
    )(ue, iv, P, combine_w)


def kernel(ufeat, ifeat, enc_src, enc_dst, enc_etype, dec_src, dec_dst,
           W, fc_w, fc_b, gn_weight, gn_bias, gn_mean_scale, P, combine_w):
    i32 = jnp.int32
    enc_src = enc_src.astype(i32)
    enc_dst = enc_dst.astype(i32)
    enc_etype = enc_etype.astype(i32)
    dec_src = dec_src.astype(i32)
    dec_dst = dec_dst.astype(i32)

    didx = jnp.stack([enc_src.reshape(16, ENC_PER_TILE),
                      enc_dst.reshape(16, ENC_PER_TILE)])
    g0 = enc_etype * N_USERS + enc_src
    g1 = NUM_RATINGS * N_USERS + enc_etype * N_ITEMS + enc_dst
    gidx = (jnp.stack([g0, g1]).reshape(1, 2, MSG_NSUP, MSG_SUP) * 16
            + jnp.arange(16, dtype=i32).reshape(16, 1, 1, 1))
    sidx = (jnp.stack([enc_dst, enc_src]) * 16).reshape(2, MSG_NSUP, MSG_SUP)
    npad = E_DEC_PAD - E_DEC
    padv = (jnp.arange(npad, dtype=i32) * 37) % N_USERS
    dsidx = jnp.concatenate([dec_src, padv]).reshape(32, DEC_PER_W)
    ddidx = jnp.concatenate([dec_dst, padv]).reshape(32, DEC_PER_W)

    feats = jnp.stack([ufeat, ifeat])

    deg3 = _sc_degrees(didx)
    dcol3 = deg3.transpose(0, 2, 1)
    htab = _tc_prep(feats, dcol3, W)
    htabV = htab.reshape(2 * NUM_RATINGS * N_USERS * 16, 16)
    agg = _sc_messages(htabV, gidx, sidx)
    aggT = (agg.reshape(2, 16, NPAD, 16)
            .transpose(0, 2, 1, 3).reshape(2, NPAD, AGG_UNITS))
    nutab, ivtab = _tc_mid(aggT, dcol3, fc_w, fc_b,
                           gn_weight, gn_bias, gn_mean_scale)
    ue, iv = _sc_dec_gather(nutab, ivtab, dsidx, ddidx)
    return _tc_final(ue, iv, P, combine_w)[:E_DEC]


# speedup vs baseline: 1.1440x; 1.1440x over previous
import functools

import jax
import jax.numpy as jnp
from jax import lax
from jax.experimental import pallas as pl
from jax.experimental.pallas import tpu as pltpu
from jax.experimental.pallas import tpu_sc as plsc

N_USERS = 5000
N_ITEMS = 5000
NUM_RATINGS = 5
D_IN = 128
AGG_UNITS = 256
OUT_UNITS = 128
NUM_BASIS = 2
E_ENC = 160000
E_DEC = 100000

NPAD = 5120
NLANE = 16
ACC_W = NPAD * NLANE

ENC_PER_TILE = E_ENC // 16

MSG_CH = 640
MSG_SUP = 6400
MSG_CHPS = MSG_SUP // MSG_CH
MSG_NSUP = E_ENC // MSG_SUP

E_DEC_PAD = 102400
DEC_PER_W = E_DEC_PAD // 32
DEC_CHUNK = 128
DEC_NCHUNK = DEC_PER_W // DEC_CHUNK

_SC_MESH = plsc.VectorSubcoreMesh(core_axis_name="c", subcore_axis_name="s")


@functools.partial(
    pl.kernel,
    out_type=jax.ShapeDtypeStruct((2, 16, NPAD), jnp.float32),
    mesh=_SC_MESH,
    scratch_types=[
        pltpu.VMEM((ENC_PER_TILE,), jnp.int32),
        pltpu.VMEM((ACC_W,), jnp.float32),
        pltpu.VMEM((NPAD,), jnp.float32),
    ],
    compiler_params=pltpu.CompilerParams(needs_layout_passes=False),
)
def _sc_degrees(didx_hbm, out_hbm, idx_v, acc_v, red_v):
    cid = lax.axis_index("c")
    tid = lax.axis_index("s")
    zeros = jnp.zeros((16,), jnp.float32)
    ones = jnp.ones((16,), jnp.float32)
    iota = lax.iota(jnp.int32, 16)

    @plsc.parallel_loop(0, ACC_W // 16, unroll=8)
    def zbody(i):
        acc_v[pl.ds(i * 16, 16)] = zeros

    pltpu.sync_copy(didx_hbm.at[cid, tid], idx_v)

    @plsc.parallel_loop(0, ENC_PER_TILE // 16, unroll=8)
    def body(g):
        s16 = idx_v[pl.ds(g * 16, 16)]
        plsc.addupdate_scatter(acc_v, [iota * NPAD + s16], ones)

    @plsc.parallel_loop(0, NPAD // 16, unroll=2)
    def rbody(i):
        v = acc_v[pl.ds(i * 16, 16)]
        for l in range(1, 16):
            v = v + acc_v[pl.ds(l * NPAD + i * 16, 16)]
        red_v[pl.ds(i * 16, 16)] = v

    pltpu.sync_copy(red_v, out_hbm.at[cid, tid])


@functools.partial(
    pl.kernel,
    out_type=jax.ShapeDtypeStruct((2, 16, ACC_W), jnp.float32),
    mesh=_SC_MESH,
    scratch_types=[
        pltpu.VMEM((MSG_SUP,), jnp.int32),
        pltpu.VMEM((MSG_SUP,), jnp.int32),
        pltpu.VMEM((MSG_CH, NLANE), jnp.float32),
        pltpu.VMEM((MSG_CH, NLANE), jnp.float32),
        pltpu.VMEM((ACC_W,), jnp.float32),
        pltpu.SemaphoreType.DMA,
        pltpu.SemaphoreType.DMA,
    ],
    compiler_params=pltpu.CompilerParams(needs_layout_passes=False,
                                         use_tc_tiling_on_sc=False),
)
def _sc_messages(htab_hbm, gidx_hbm, sidx_hbm, out_hbm,
                 gv, sv, rows0, rows1, acc_v, sem0, sem1):
    cid = lax.axis_index("c")
    tid = lax.axis_index("s")
    zeros = jnp.zeros((16,), jnp.float32)
    iota = lax.iota(jnp.int32, 16)

    @plsc.parallel_loop(0, ACC_W // 16, unroll=8)
    def zbody(i):
        acc_v[pl.ds(i * 16, 16)] = zeros

    tab = htab_hbm
    bufs = (rows0, rows1)
    sems = (sem0, sem1)

    def gather(c):
        return pltpu.async_copy(
            tab.at[gv.at[pl.ds(c * MSG_CH, MSG_CH)]],
            bufs[c % 2], sems[c % 2])

    def accumulate(c):
        rows = bufs[c % 2]

        @plsc.parallel_loop(0, MSG_CH // 16, unroll=4)
        def abody(g):
            base = c * MSG_CH + g * 16
            d16 = sv[pl.ds(base, 16)]
            for u in range(16):
                plsc.addupdate_scatter(
                    acc_v, [d16[u] + iota], rows[g * 16 + u])

    def sup(b, _):
        pltpu.sync_copy(gidx_hbm.at[tid, cid, b], gv)
        pltpu.sync_copy(sidx_hbm.at[cid, b], sv)
        cps = [gather(0)]
        for c in range(MSG_CHPS):
            if c + 1 < MSG_CHPS:
                cps.append(gather(c + 1))
            cps[c].wait()
            accumulate(c)
        return _

    lax.fori_loop(0, MSG_NSUP, sup, None)
    pltpu.sync_copy(acc_v, out_hbm.at[cid, tid])


@functools.partial(
    pl.kernel,
    out_type=(jax.ShapeDtypeStruct((E_DEC_PAD, OUT_UNITS), jnp.float32),
              jax.ShapeDtypeStruct((E_DEC_PAD, OUT_UNITS), jnp.float32)),
    mesh=_SC_MESH,
    scratch_types=[
        pltpu.VMEM((DEC_PER_W,), jnp.int32),
        pltpu.VMEM((DEC_PER_W,), jnp.int32),
        pltpu.VMEM((DEC_CHUNK, OUT_UNITS), jnp.float32),
        pltpu.VMEM((DEC_CHUNK, OUT_UNITS), jnp.float32),
        pltpu.SemaphoreType.DMA,
        pltpu.SemaphoreType.DMA,
    ],
)
def _sc_dec_gather(nutab_hbm, ivtab_hbm, sidx_hbm, didx_hbm,
                   ue_hbm, iv_hbm, sidx_v, didx_v, u_v, v_v, sem1, sem2):
    cid = lax.axis_index("c")
    tid = lax.axis_index("s")
    wid = tid * 2 + cid
    pltpu.sync_copy(sidx_hbm.at[wid], sidx_v)
    pltpu.sync_copy(didx_hbm.at[wid], didx_v)
    base = wid * DEC_PER_W

    def body(j, _):
        cu = pltpu.async_copy(
            nutab_hbm.at[sidx_v.at[pl.ds(j * DEC_CHUNK, DEC_CHUNK)]],
            u_v, sem1)
        cv = pltpu.async_copy(
            ivtab_hbm.at[didx_v.at[pl.ds(j * DEC_CHUNK, DEC_CHUNK)]],
            v_v, sem2)
        cu.wait()
        cv.wait()
        pltpu.sync_copy(u_v, ue_hbm.at[pl.ds(base + j * DEC_CHUNK, DEC_CHUNK)])
        pltpu.sync_copy(v_v, iv_hbm.at[pl.ds(base + j * DEC_CHUNK, DEC_CHUNK)])
        return _

    lax.fori_loop(0, DEC_NCHUNK, body, None)


def _prep_body(feat_ref, deg_ref, w_ref, out_ref):
    d = jnp.sum(deg_ref[0], axis=1, keepdims=True)
    c = lax.rsqrt(jnp.maximum(d[:N_USERS, :], 1.0))
    x = feat_ref[0] * c
    out_ref[...] = jnp.dot(x, w_ref[0], preferred_element_type=jnp.float32)


def _tc_prep(feats, dcol3, W):
    return pl.pallas_call(
        _prep_body,
        grid=(2, NUM_RATINGS),
        in_specs=[
            pl.BlockSpec((1, N_USERS, D_IN), lambda s, r: (s, 0, 0)),
            pl.BlockSpec((1, NPAD, 16), lambda s, r: (s, 0, 0)),
            pl.BlockSpec((1, D_IN, AGG_UNITS), lambda s, r: (r, 0, 0)),
        ],
        out_specs=pl.BlockSpec((N_USERS, AGG_UNITS),
                               lambda s, r: (s * NUM_RATINGS + r, 0)),
        out_shape=jax.ShapeDtypeStruct((2 * NUM_RATINGS * N_USERS, AGG_UNITS),
                                       jnp.float32),
    )(feats, dcol3, W)


def _mid_body(agg_ref, deg_ref, fcw_ref, fcb_ref, gnw_ref, gnb_ref,
              gms_ref, nu_ref, iv_ref):
    du = jnp.sum(deg_ref[0], axis=1, keepdims=True)
    di = jnp.sum(deg_ref[1], axis=1, keepdims=True)
    cu = lax.rsqrt(jnp.maximum(du[:N_USERS, :], 1.0))
    ci = lax.rsqrt(jnp.maximum(di[:N_ITEMS, :], 1.0))
    ua = agg_ref[1, :N_USERS, :] * cu
    ia = agg_ref[0, :N_ITEMS, :] * ci
    ua = jnp.where(ua >= 0, ua, 0.1 * ua)
    ia = jnp.where(ia >= 0, ia, 0.1 * ia)
    uo = jnp.dot(ua, fcw_ref[...], preferred_element_type=jnp.float32) + fcb_ref[...]
    io = jnp.dot(ia, fcw_ref[...], preferred_element_type=jnp.float32) + fcb_ref[...]
    n_tot = float(N_USERS + N_ITEMS)
    mean = (jnp.sum(uo, axis=0, keepdims=True)
            + jnp.sum(io, axis=0, keepdims=True)) / n_tot
    subu = uo - mean * gms_ref[...]
    subi = io - mean * gms_ref[...]
    var = (jnp.sum(subu * subu, axis=0, keepdims=True)
           + jnp.sum(subi * subi, axis=0, keepdims=True)) / n_tot
    inv_std = lax.rsqrt(var + 1e-6)
    nu = gnw_ref[...] * subu * inv_std + gnb_ref[...]
    ni = gnw_ref[...] * subi * inv_std + gnb_ref[...]
    nu_ref[...] = nu
    iv_ref[...] = ni


def _tc_mid(agg, dcol3, fc_w, fc_b, gn_weight, gn_bias, gn_mean_scale):
    return pl.pallas_call(
        _mid_body,
        out_shape=(jax.ShapeDtypeStruct((N_USERS, OUT_UNITS), jnp.float32),
                   jax.ShapeDtypeStruct((N_ITEMS, OUT_UNITS), jnp.float32)),
    )(agg, dcol3, fc_w, fc_b[None, :], gn_weight[None, :], gn_bias[None, :],
      gn_mean_scale[None, :])


_FIN_BLK = 2048


def _final_body(ue_ref, iv_ref, p_ref, cw_ref, out_ref):
    ue = ue_ref[...]
    iv = iv_ref[...]
    t0 = lax.dot_general(iv, p_ref[0], (((1,), (1,)), ((), ())),
                         preferred_element_type=jnp.float32)
    t1 = lax.dot_general(iv, p_ref[1], (((1,), (1,)), ((), ())),
                         preferred_element_type=jnp.float32)
    b0 = jnp.sum(ue * t0, axis=1, keepdims=True)
    b1 = jnp.sum(ue * t1, axis=1, keepdims=True)
    out_ref[...] = b0 * cw_ref[0, :][None, :] + b1 * cw_ref[1, :][None, :]


def _tc_final(ue, iv, P, combine_w):
    return pl.pallas_call(
        _final_body,
        grid=(E_DEC_PAD // _FIN_BLK,),
        in_specs=[
            pl.BlockSpec((_FIN_BLK, OUT_UNITS), lambda i: (i, 0)),
            pl.BlockSpec((_FIN_BLK, OUT_UNITS), lambda i: (i, 0)),
            pl.BlockSpec((NUM_BASIS, OUT_UNITS, OUT_UNITS), lambda i: (0, 0, 0)),
            pl.BlockSpec((NUM_BASIS, NUM_RATINGS), lambda i: (0, 0)),
        ],
        out_specs=pl.BlockSpec((_FIN_BLK, NUM_RATINGS), lambda i: (i, 0)),
        out_shape=jax.ShapeDtypeStruct((E_DEC_PAD, NUM_RATINGS), jnp.float32),
    )(ue, iv, P, combine_w)


def kernel(ufeat, ifeat, enc_src, enc_dst, enc_etype, dec_src, dec_dst,
           W, fc_w, fc_b, gn_weight, gn_bias, gn_mean_scale, P, combine_w):
    i32 = jnp.int32
    enc_src = enc_src.astype(i32)
    enc_dst = enc_dst.astype(i32)
    enc_etype = enc_etype.astype(i32)
    dec_src = dec_src.astype(i32)
    dec_dst = dec_dst.astype(i32)

    didx = jnp.stack([enc_src.reshape(16, ENC_PER_TILE),
                      enc_dst.reshape(16, ENC_PER_TILE)])
    g0 = enc_etype * N_USERS + enc_src
    g1 = NUM_RATINGS * N_USERS + enc_etype * N_ITEMS + enc_dst
    gidx = (jnp.stack([g0, g1]).reshape(1, 2, MSG_NSUP, MSG_SUP) * 16
            + jnp.arange(16, dtype=i32).reshape(16, 1, 1, 1))
    sidx = (jnp.stack([enc_dst, enc_src]) * 16).reshape(2, MSG_NSUP, MSG_SUP)
    npad = E_DEC_PAD - E_DEC
    padv = (jnp.arange(npad, dtype=i32) * 37) % N_USERS
    dsidx = jnp.concatenate([dec_src, padv]).reshape(32, DEC_PER_W)
    ddidx = jnp.concatenate([dec_dst, padv]).reshape(32, DEC_PER_W)

    feats = jnp.stack([ufeat, ifeat])

    deg3 = _sc_degrees(didx)
    dcol3 = deg3.transpose(0, 2, 1)
    htab = _tc_prep(feats, dcol3, W)
    htabV = htab.reshape(2 * NUM_RATINGS * N_USERS * 16, 16)
    agg = _sc_messages(htabV, gidx, sidx)
    aggT = (agg.reshape(2, 16, NPAD, 16)
            .transpose(0, 2, 1, 3).reshape(2, NPAD, AGG_UNITS))
    nutab, ivtab = _tc_mid(aggT, dcol3, fc_w, fc_b,
                           gn_weight, gn_bias, gn_mean_scale)
    ue, iv = _sc_dec_gather(nutab, ivtab, dsidx, ddidx)
    return _tc_final(ue, iv, P, combine_w)[:E_DEC]
